# baseline (speedup 1.0000x reference)
import functools

import jax
import jax.numpy as jnp
from jax import lax
from jax.experimental import pallas as pl
from jax.experimental.pallas import tpu as pltpu
from jax.experimental.pallas import tpu_sc as plsc

D_MODEL = 128
NSLOT = 4


@functools.lru_cache(maxsize=None)
def _num_workers():
    info = plsc.get_sparse_core_info()
    return info.num_cores, info.num_subcores


def _add_kernel(a_ref, b_ref, o_ref):
    o_ref[...] = a_ref[...] + b_ref[...]


def _sum_tables(a, b):
    n, d = a.shape
    blk = 2000
    grid = n // blk
    return pl.pallas_call(
        _add_kernel,
        out_shape=jax.ShapeDtypeStruct((n, d), a.dtype),
        grid=(grid,),
        in_specs=[
            pl.BlockSpec((blk, d), lambda i: (i, 0)),
            pl.BlockSpec((blk, d), lambda i: (i, 0)),
        ],
        out_specs=pl.BlockSpec((blk, d), lambda i: (i, 0)),
    )(a, b)


@functools.lru_cache(maxsize=None)
def _make_gather(n_batch, s, d):
    nc, ns = _num_workers()
    nw = nc * ns
    per_w = n_batch // nw
    n_rounds = per_w // NSLOT

    mesh = plsc.VectorSubcoreMesh(core_axis_name="c", subcore_axis_name="s")

    @functools.partial(
        pl.kernel,
        out_type=jax.ShapeDtypeStruct((n_batch, s, d), jnp.float32),
        mesh=mesh,
        compiler_params=pltpu.CompilerParams(use_tc_tiling_on_sc=True),
        scratch_types=[
            pltpu.VMEM((per_w, d), jnp.int32),
            [pltpu.VMEM((s, d), jnp.float32) for _ in range(NSLOT)],
            [pltpu.SemaphoreType.DMA for _ in range(NSLOT)],
            [pltpu.SemaphoreType.DMA for _ in range(NSLOT)],
        ],
    )
    def gather_kernel(x_hbm, tab_hbm, out_hbm, idx_v, bufs, gsems, wsems):
        wid = lax.axis_index("s") * nc + lax.axis_index("c")
        b0 = wid * per_w
        pltpu.sync_copy(x_hbm.at[wid], idx_v)

        def fire_gather(bb, k):
            pltpu.async_copy(
                tab_hbm.at[idx_v.at[bb, pl.ds(0, s)]], bufs[k], gsems[k]
            )

        def wait_gather(k):
            pltpu.make_async_copy(
                tab_hbm.at[idx_v.at[0, pl.ds(0, s)]], bufs[k], gsems[k]
            ).wait()

        def fire_write(bb, k):
            pltpu.async_copy(bufs[k], out_hbm.at[b0 + bb], wsems[k])

        def wait_write(k):
            pltpu.make_async_copy(bufs[k], out_hbm.at[0], wsems[k]).wait()

        for k in range(NSLOT):
            fire_gather(k, k)

        def body(i, carry):
            base = i * NSLOT
            for k in range(NSLOT):
                wait_gather(k)
                fire_write(base + k, k)

            @pl.when(i < n_rounds - 1)
            def _refill():
                for k in range(NSLOT):
                    wait_write(k)
                    fire_gather(base + NSLOT + k, k)

            return carry

        lax.fori_loop(0, n_rounds, body, 0)
        for k in range(NSLOT):
            wait_write(k)

    return gather_kernel


def kernel(x, token_table, pos_table):
    summed = _sum_tables(token_table, pos_table)
    b, s = x.shape
    nc, ns = _num_workers()
    nw = nc * ns
    xp = jnp.pad(x.astype(jnp.int32), ((0, 0), (0, D_MODEL - s)))
    x3d = xp.reshape(nw, b // nw, D_MODEL)
    out = _make_gather(b, s, D_MODEL)(x3d, summed)
    return out


# speedup vs baseline: 1.0648x; 1.0648x over previous
import functools

import jax
import jax.numpy as jnp
from jax import lax
from jax.experimental import pallas as pl
from jax.experimental.pallas import tpu as pltpu
from jax.experimental.pallas import tpu_sc as plsc

D_MODEL = 128
NSLOT = 8


@functools.lru_cache(maxsize=None)
def _num_workers():
    info = plsc.get_sparse_core_info()
    return info.num_cores, info.num_subcores


def _add_kernel(a_ref, b_ref, o_ref):
    o_ref[...] = a_ref[...] + b_ref[...]


def _sum_tables(a, b):
    n, d = a.shape
    blk = 4000
    grid = n // blk
    return pl.pallas_call(
        _add_kernel,
        out_shape=jax.ShapeDtypeStruct((n, d), a.dtype),
        grid=(grid,),
        in_specs=[
            pl.BlockSpec((blk, d), lambda i: (i, 0)),
            pl.BlockSpec((blk, d), lambda i: (i, 0)),
        ],
        out_specs=pl.BlockSpec((blk, d), lambda i: (i, 0)),
    )(a, b)


@functools.lru_cache(maxsize=None)
def _make_gather(n_batch, s, d):
    nc, ns = _num_workers()
    nw = nc * ns
    per_w = n_batch // nw
    n_rounds = per_w // NSLOT

    mesh = plsc.VectorSubcoreMesh(core_axis_name="c", subcore_axis_name="s")

    @functools.partial(
        pl.kernel,
        out_type=jax.ShapeDtypeStruct((n_batch, s, d), jnp.float32),
        mesh=mesh,
        compiler_params=pltpu.CompilerParams(use_tc_tiling_on_sc=True),
        scratch_types=[
            pltpu.VMEM((per_w, d), jnp.int32),
            [pltpu.VMEM((s, d), jnp.float32) for _ in range(NSLOT)],
            [pltpu.SemaphoreType.DMA for _ in range(NSLOT)],
            [pltpu.SemaphoreType.DMA for _ in range(NSLOT)],
        ],
    )
    def gather_kernel(x_hbm, tab_hbm, out_hbm, idx_v, bufs, gsems, wsems):
        wid = lax.axis_index("s") * nc + lax.axis_index("c")
        b0 = wid * per_w
        pltpu.sync_copy(x_hbm.at[wid], idx_v)

        def fire_gather(bb, k):
            pltpu.async_copy(
                tab_hbm.at[idx_v.at[bb, pl.ds(0, s)]], bufs[k], gsems[k]
            )

        def wait_gather(k):
            pltpu.make_async_copy(
                tab_hbm.at[idx_v.at[0, pl.ds(0, s)]], bufs[k], gsems[k]
            ).wait()

        def fire_write(bb, k):
            pltpu.async_copy(bufs[k], out_hbm.at[b0 + bb], wsems[k])

        def wait_write(k):
            pltpu.make_async_copy(bufs[k], out_hbm.at[0], wsems[k]).wait()

        for k in range(NSLOT):
            fire_gather(k, k)

        def body(i, carry):
            base = i * NSLOT
            for k in range(NSLOT):
                wait_gather(k)
                fire_write(base + k, k)

            @pl.when(i < n_rounds - 1)
            def _refill():
                for k in range(NSLOT):
                    wait_write(k)
                    fire_gather(base + NSLOT + k, k)

            return carry

        lax.fori_loop(0, n_rounds, body, 0)
        for k in range(NSLOT):
            wait_write(k)

    return gather_kernel


def kernel(x, token_table, pos_table):
    summed = _sum_tables(token_table, pos_table)
    b, s = x.shape
    nc, ns = _num_workers()
    nw = nc * ns
    xp = jnp.pad(x.astype(jnp.int32), ((0, 0), (0, D_MODEL - s)))
    x3d = xp.reshape(nw, b // nw, D_MODEL)
    out = _make_gather(b, s, D_MODEL)(x3d, summed)
    return out
